# Initial kernel scaffold; baseline (speedup 1.0000x reference)
import jax
import jax.numpy as jnp
from jax.experimental import pallas as pl


def kernel(x, edge_index, percent, ricci_curvature, W1, b1, W2, b2, W3, b3):
    raise NotImplementedError("write your pallas kernel here")


# speedup vs baseline: 8.5214x; 8.5214x over previous
import functools

import jax
import jax.numpy as jnp
from jax import lax
from jax.experimental import pallas as pl
from jax.experimental.pallas import tpu as pltpu
from jax.experimental.pallas import tpu_sc as plsc

N = 10000
D = 128
NPAD = 10240
PAD_ROW = NPAD - 1
NC = 2
NS = 16
NW = NC * NS
CH = 128
ROWS_PER_TILE = NPAD // NS
BLK = 512


def _cdiv(a, b):
    return (a + b - 1) // b


def _sc_mesh():
    return plsc.VectorSubcoreMesh(core_axis_name="c", subcore_axis_name="s")


@functools.partial(jax.jit, static_argnames=("cpw",))
def _sc_degree(dst_p, zerosf, onesf, *, cpw):
    @functools.partial(
        pl.kernel,
        out_type=jax.ShapeDtypeStruct((NC, NPAD, D), jnp.float32),
        mesh=_sc_mesh(),
        scratch_types=[
            pltpu.VMEM((CH,), jnp.int32),
            pltpu.VMEM((CH, D), jnp.float32),
            pltpu.VMEM_SHARED((NPAD, D), jnp.float32),
        ],
    )
    def deg_kernel(dst_hbm, z_hbm, ones_hbm, out_hbm, dst_v, ones_v, acc_sh):
        c = lax.axis_index("c")
        s = lax.axis_index("s")
        wid = c * NS + s
        r0 = s * ROWS_PER_TILE
        pltpu.sync_copy(z_hbm.at[pl.ds(r0, ROWS_PER_TILE)],
                        acc_sh.at[pl.ds(r0, ROWS_PER_TILE)])
        pltpu.sync_copy(ones_hbm, ones_v)
        plsc.subcore_barrier()

        def body(j, carry):
            base = pl.multiple_of(wid * (cpw * CH) + j * CH, CH)
            pltpu.sync_copy(dst_hbm.at[pl.ds(base, CH)], dst_v)
            pltpu.sync_copy(ones_v, acc_sh.at[dst_v], add=True)
            return carry

        lax.fori_loop(0, cpw, body, 0)
        plsc.subcore_barrier()
        pltpu.sync_copy(acc_sh.at[pl.ds(r0, ROWS_PER_TILE)],
                        out_hbm.at[c, pl.ds(r0, ROWS_PER_TILE)])

    return deg_kernel(dst_p, zerosf, onesf)


@functools.partial(jax.jit, static_argnames=("cpw",))
def _sc_scatter(src_p, dst_p, g, zerosf, *, cpw):
    @functools.partial(
        pl.kernel,
        out_type=jax.ShapeDtypeStruct((NC, NPAD, D), jnp.float32),
        mesh=_sc_mesh(),
        scratch_types=[
            pltpu.VMEM((CH,), jnp.int32),
            pltpu.VMEM((CH,), jnp.int32),
            pltpu.VMEM((CH, D), jnp.float32),
            pltpu.VMEM_SHARED((NPAD, D), jnp.float32),
            pltpu.SemaphoreType.DMA,
        ],
    )
    def msg_kernel(src_hbm, dst_hbm, g_hbm, z_hbm, out_hbm,
                   src_v, dst_v, rows_v, acc_sh, sem):
        c = lax.axis_index("c")
        s = lax.axis_index("s")
        wid = c * NS + s
        r0 = s * ROWS_PER_TILE
        pltpu.sync_copy(z_hbm.at[pl.ds(r0, ROWS_PER_TILE)],
                        acc_sh.at[pl.ds(r0, ROWS_PER_TILE)])
        plsc.subcore_barrier()

        def body(j, carry):
            base = pl.multiple_of(wid * (cpw * CH) + j * CH, CH)
            pltpu.sync_copy(src_hbm.at[pl.ds(base, CH)], src_v)
            pltpu.sync_copy(dst_hbm.at[pl.ds(base, CH)], dst_v)
            pltpu.async_copy(g_hbm.at[src_v], rows_v, sem).wait()
            pltpu.sync_copy(rows_v, acc_sh.at[dst_v], add=True)
            return carry

        lax.fori_loop(0, cpw, body, 0)
        plsc.subcore_barrier()
        pltpu.sync_copy(acc_sh.at[pl.ds(r0, ROWS_PER_TILE)],
                        out_hbm.at[c, pl.ds(r0, ROWS_PER_TILE)])

    return msg_kernel(src_p, dst_p, g, zerosf)


_HI = jax.lax.Precision.HIGHEST


def _dinv_g1_body(dega_ref, degb_ref, x_ref, w_ref, dinv_ref, g_ref):
    i = pl.program_id(0)
    deg = dega_ref[...][:, :1] + degb_ref[...][:, :1] + 1.0
    rows = i * BLK + lax.broadcasted_iota(jnp.int32, (BLK, 1), 0)
    dinv = jnp.where(rows < N, lax.rsqrt(deg), 0.0)
    dinvm = jnp.broadcast_to(dinv, (BLK, D))
    dinv_ref[...] = dinvm
    g_ref[...] = jnp.dot(x_ref[...], w_ref[...],
                         preferred_element_type=jnp.float32,
                         precision=_HI) * dinvm


@jax.jit
def _tc_dinv_g1(dega, degb, x_p, W1):
    grid = (NPAD // BLK,)
    return pl.pallas_call(
        _dinv_g1_body,
        grid=grid,
        in_specs=[
            pl.BlockSpec((BLK, D), lambda i: (i, 0)),
            pl.BlockSpec((BLK, D), lambda i: (i, 0)),
            pl.BlockSpec((BLK, D), lambda i: (i, 0)),
            pl.BlockSpec((D, D), lambda i: (0, 0)),
        ],
        out_specs=[
            pl.BlockSpec((BLK, D), lambda i: (i, 0)),
            pl.BlockSpec((BLK, D), lambda i: (i, 0)),
        ],
        out_shape=[
            jax.ShapeDtypeStruct((NPAD, D), jnp.float32),
            jax.ShapeDtypeStruct((NPAD, D), jnp.float32),
        ],
    )(dega, degb, x_p, W1)


def _layer_body(acca_ref, accb_ref, g_ref, dinv_ref, b_ref, xprev_ref,
                wa_ref, wb_ref, xo_ref, go_ref):
    dm = dinv_ref[...]
    xn = jnp.maximum((acca_ref[...] + accb_ref[...] + g_ref[...]) * dm
                     + b_ref[...], 0.0)
    xo_ref[...] = xn
    go_ref[...] = (jnp.dot(xn, wa_ref[...],
                           preferred_element_type=jnp.float32, precision=_HI)
                   + jnp.dot(xprev_ref[...], wb_ref[...],
                             preferred_element_type=jnp.float32,
                             precision=_HI)) * dm


@jax.jit
def _tc_layer(acca, accb, g, dinvm, b, xprev, Wa, Wb):
    grid = (NPAD // BLK,)
    blk = pl.BlockSpec((BLK, D), lambda i: (i, 0))
    return pl.pallas_call(
        _layer_body,
        grid=grid,
        in_specs=[
            blk, blk, blk, blk,
            pl.BlockSpec((1, D), lambda i: (0, 0)),
            blk,
            pl.BlockSpec((D, D), lambda i: (0, 0)),
            pl.BlockSpec((D, D), lambda i: (0, 0)),
        ],
        out_specs=[blk, blk],
        out_shape=[
            jax.ShapeDtypeStruct((NPAD, D), jnp.float32),
            jax.ShapeDtypeStruct((NPAD, D), jnp.float32),
        ],
    )(acca, accb, g, dinvm, b, xprev, Wa, Wb)


def _final_body(acca_ref, accb_ref, g_ref, dinv_ref, b_ref, o_ref):
    o_ref[...] = ((acca_ref[...] + accb_ref[...] + g_ref[...])
                  * dinv_ref[...] + b_ref[...])


@jax.jit
def _tc_final(acca, accb, g, dinvm, b):
    grid = (NPAD // BLK,)
    blk = pl.BlockSpec((BLK, D), lambda i: (i, 0))
    return pl.pallas_call(
        _final_body,
        grid=grid,
        in_specs=[blk, blk, blk, blk, pl.BlockSpec((1, D), lambda i: (0, 0))],
        out_specs=blk,
        out_shape=jax.ShapeDtypeStruct((NPAD, D), jnp.float32),
    )(acca, accb, g, dinvm, b)


def kernel(x, edge_index, percent, ricci_curvature, W1, b1, W2, b2, W3, b3):
    del percent, ricci_curvature
    E = edge_index.shape[1]
    cpw = _cdiv(E, NW * CH)
    EPAD = NW * cpw * CH

    pad = jnp.full((EPAD - E,), PAD_ROW, dtype=edge_index.dtype)
    src_p = jnp.concatenate([edge_index[0], pad])
    dst_p = jnp.concatenate([edge_index[1], pad])
    x_p = jnp.pad(x, ((0, NPAD - N), (0, 0)))
    onesf = jnp.ones((CH, D), jnp.float32)
    zerosf = jnp.zeros((NPAD, D), jnp.float32)

    degp = _sc_degree(dst_p, zerosf, onesf, cpw=cpw)
    dinvm, g1 = _tc_dinv_g1(degp[0], degp[1], x_p, W1)

    acc1 = _sc_scatter(src_p, dst_p, g1, zerosf, cpw=cpw)
    x1, g2 = _tc_layer(acc1[0], acc1[1], g1, dinvm, b1.reshape(1, D),
                       x_p, W2[:D], W2[D:])

    acc2 = _sc_scatter(src_p, dst_p, g2, zerosf, cpw=cpw)
    x2, g3 = _tc_layer(acc2[0], acc2[1], g2, dinvm, b2.reshape(1, D),
                       x1, W3[:D], W3[D:])

    acc3 = _sc_scatter(src_p, dst_p, g3, zerosf, cpw=cpw)
    out = _tc_final(acc3[0], acc3[1], g3, dinvm, b3.reshape(1, D))

    return out[:N], x1[:N], x2[:N]
